# Initial kernel scaffold; baseline (speedup 1.0000x reference)
import jax
import jax.numpy as jnp
from jax.experimental import pallas as pl


def kernel(output, y):
    raise NotImplementedError("write your pallas kernel here")


# speedup vs baseline: 3.2830x; 3.2830x over previous
import functools

import jax
import jax.numpy as jnp
from jax import lax
from jax.experimental import pallas as pl
from jax.experimental.pallas import tpu as pltpu

B = 16384
C = 1000
R = 512


def _dense_body(x_ref, y_ref, o_ref):
    x = x_ref[...]
    y = y_ref[...]
    cols = lax.broadcasted_iota(jnp.int32, (R, C), 1)
    onehot = (cols == y).astype(jnp.float32)
    s = jnp.sum(x * onehot, axis=1, keepdims=True)
    t = jnp.maximum(x - s + 1.0, 0.0)
    o_ref[...] = (jnp.sum(t, axis=1) - 1.0) * (1.0 / C)


def kernel(output, y):
    grid = (B // R,)
    y_col = y.reshape(B, 1)
    return pl.pallas_call(
        _dense_body,
        grid=grid,
        in_specs=[
            pl.BlockSpec((R, C), lambda i: (i, 0)),
            pl.BlockSpec((R, 1), lambda i: (i, 0)),
        ],
        out_specs=pl.BlockSpec((R,), lambda i: (i,)),
        out_shape=jax.ShapeDtypeStruct((B,), jnp.float32),
    )(output, y_col)


# speedup vs baseline: 3.5263x; 1.0741x over previous
import functools

import jax
import jax.numpy as jnp
from jax import lax
from jax.experimental import pallas as pl
from jax.experimental.pallas import tpu as pltpu

B = 16384
C = 1000
R = 1024


def _dense_body(x_ref, y_ref, o_ref):
    x = x_ref[...]
    y = y_ref[...]
    cols = lax.broadcasted_iota(jnp.int32, (R, C), 1)
    onehot = (cols == y).astype(jnp.float32)
    s = jnp.sum(x * onehot, axis=1, keepdims=True)
    t = jnp.maximum(x - s + 1.0, 0.0)
    o_ref[...] = (jnp.sum(t, axis=1) - 1.0) * (1.0 / C)


def kernel(output, y):
    grid = (B // R,)
    y_col = y.reshape(B, 1)
    return pl.pallas_call(
        _dense_body,
        grid=grid,
        in_specs=[
            pl.BlockSpec((R, C), lambda i: (i, 0)),
            pl.BlockSpec((R, 1), lambda i: (i, 0)),
        ],
        out_specs=pl.BlockSpec((R,), lambda i: (i,)),
        out_shape=jax.ShapeDtypeStruct((B,), jnp.float32),
    )(output, y_col)
